# baseline (speedup 1.0000x reference)
import functools

import jax
import jax.numpy as jnp
from jax import lax
from jax.experimental import pallas as pl
from jax.experimental.pallas import tpu as pltpu
from jax.experimental.pallas import tpu_sc as plsc

_NC = 2
_NS = 16
_NW = _NC * _NS


@functools.partial(jax.jit, static_argnums=(2, 3))
def _sc_gather(idx, table, B, D):
    b_per_w = B // _NW
    C = 64
    n_chunks = b_per_w // C

    mesh = plsc.VectorSubcoreMesh(core_axis_name="c", subcore_axis_name="s")

    @functools.partial(
        pl.kernel,
        mesh=mesh,
        out_type=jax.ShapeDtypeStruct((B, D), jnp.float32),
        scratch_types=[
            pltpu.VMEM((b_per_w,), jnp.int32),
            pltpu.VMEM((C, D), jnp.float32),
            pltpu.VMEM((C, D), jnp.float32),
            pltpu.SemaphoreType.DMA,
            pltpu.SemaphoreType.DMA,
            pltpu.SemaphoreType.DMA,
            pltpu.SemaphoreType.DMA,
        ],
    )
    def k(idx_hbm, table_hbm, out_hbm, idx_v, rows0, rows1, g0, g1, s0, s1):
        wid = lax.axis_index("s") * _NC + lax.axis_index("c")
        base = wid * b_per_w
        pltpu.sync_copy(idx_hbm.at[pl.ds(base, b_per_w)], idx_v)
        bufs = (rows0, rows1)
        gsems = (g0, g1)
        ssems = (s0, s1)
        gathers = [None] * n_chunks
        scatters = [None] * n_chunks
        for g in range(n_chunks):
            p = g % 2
            if g >= 2:
                scatters[g - 2].wait()
            gathers[g] = pltpu.async_copy(
                table_hbm.at[idx_v.at[pl.ds(g * C, C)]], bufs[p], gsems[p]
            )
            if g >= 1:
                q = (g - 1) % 2
                gathers[g - 1].wait()
                scatters[g - 1] = pltpu.async_copy(
                    bufs[q], out_hbm.at[pl.ds(base + (g - 1) * C, C)], ssems[q]
                )
        last = n_chunks - 1
        gathers[last].wait()
        scatters[last] = pltpu.async_copy(
            bufs[last % 2], out_hbm.at[pl.ds(base + last * C, C)], ssems[last % 2]
        )
        scatters[last - 1].wait()
        scatters[last].wait()

    return k(idx, table)


def kernel(x, W_E):
    B, S = x.shape
    V, D = W_E.shape
    flat = x.reshape(B * S).astype(jnp.int32)
    out = _sc_gather(flat, W_E, B * S, D)
    return out.reshape(B, S, D)
